# baseline (speedup 1.0000x reference)
import functools

import jax
import jax.numpy as jnp
from jax import lax
from jax.experimental import pallas as pl
from jax.experimental.pallas import tpu as pltpu
from jax.experimental.pallas import tpu_sc as plsc


_NC = 2
_NS = 16
_NW = _NC * _NS
_CHUNK = 128
_NBUF = 3


def kernel(inputs):
    n, c = inputs.shape
    rows_per_w = n // _NW
    nchunks = rows_per_w // _CHUNK
    mesh = plsc.VectorSubcoreMesh(core_axis_name="c", subcore_axis_name="s")

    @functools.partial(
        pl.kernel,
        mesh=mesh,
        out_type=[
            jax.ShapeDtypeStruct((n, c), jnp.float32),
            jax.ShapeDtypeStruct((n,), jnp.float32),
        ],
        scratch_types=[
            pltpu.VMEM((_NBUF, _CHUNK, c), jnp.float32),
            pltpu.VMEM((rows_per_w,), jnp.float32),
            pltpu.SemaphoreType.DMA((_NBUF,)),
            pltpu.SemaphoreType.DMA((_NBUF,)),
            pltpu.SemaphoreType.DMA,
        ],
    )
    def _sc_copy(x_hbm, out_hbm, ld_hbm, buf, zbuf, in_sems, out_sems, zsem):
        wid = lax.axis_index("s") * _NC + lax.axis_index("c")
        base = wid * rows_per_w

        def _in_copy(i):
            return pltpu.make_async_copy(
                x_hbm.at[pl.ds(base + i * _CHUNK, _CHUNK)],
                buf.at[i % _NBUF],
                in_sems.at[i % _NBUF],
            )

        def _out_copy(i):
            return pltpu.make_async_copy(
                buf.at[i % _NBUF],
                out_hbm.at[pl.ds(base + i * _CHUNK, _CHUNK)],
                out_sems.at[i % _NBUF],
            )

        for i in range(_NBUF):
            _in_copy(i).start()

        def _zfill(i, carry):
            zbuf[pl.ds(i * 16, 16)] = jnp.zeros((16,), jnp.float32)
            return carry

        lax.fori_loop(0, rows_per_w // 16, _zfill, 0)
        zcopy = pltpu.make_async_copy(
            zbuf, ld_hbm.at[pl.ds(base, rows_per_w)], zsem
        )
        zcopy.start()

        for i in range(nchunks):
            _in_copy(i).wait()
            _out_copy(i).start()
            j = i - 1
            if 0 <= j and j + _NBUF < nchunks:
                _out_copy(j).wait()
                _in_copy(j + _NBUF).start()

        for i in range(nchunks - _NBUF, nchunks):
            _out_copy(i).wait()
        zcopy.wait()

    outputs, ld = _sc_copy(inputs)
    return (outputs, ld)


# speedup vs baseline: 1.3657x; 1.3657x over previous
import jax
import jax.numpy as jnp
from jax.experimental import pallas as pl
from jax.experimental.pallas import tpu as pltpu


_BLOCK_ROWS = 8192


def _copy_body(x_ref, y_ref, ld_ref):
    y_ref[...] = x_ref[...]
    ld_ref[...] = jnp.zeros_like(ld_ref)


def kernel(inputs):
    n, c = inputs.shape
    block_rows = _BLOCK_ROWS if n % _BLOCK_ROWS == 0 else n
    grid = (n // block_rows,)
    outputs, ld = pl.pallas_call(
        _copy_body,
        grid=grid,
        in_specs=[pl.BlockSpec((block_rows, c), lambda i: (i, 0))],
        out_specs=[
            pl.BlockSpec((block_rows, c), lambda i: (i, 0)),
            pl.BlockSpec((block_rows,), lambda i: (i,)),
        ],
        out_shape=[
            jax.ShapeDtypeStruct((n, c), inputs.dtype),
            jax.ShapeDtypeStruct((n,), jnp.float32),
        ],
        compiler_params=pltpu.CompilerParams(
            dimension_semantics=("parallel",),
        ),
    )(inputs)
    return (outputs, ld)


# speedup vs baseline: 1.3801x; 1.0105x over previous
import jax
import jax.numpy as jnp
from jax.experimental import pallas as pl
from jax.experimental.pallas import tpu as pltpu


_BLOCK_ROWS = 16128


def _copy_body(x_ref, y_ref, ld_ref):
    y_ref[...] = x_ref[...]
    ld_ref[...] = jnp.zeros_like(ld_ref)


def kernel(inputs):
    n, c = inputs.shape
    block_rows = min(_BLOCK_ROWS, n)
    grid = (pl.cdiv(n, block_rows),)
    ld_block = 1024 * pl.cdiv(n, 1024 * grid[0])
    outputs, ld = pl.pallas_call(
        _copy_body,
        grid=grid,
        in_specs=[pl.BlockSpec((block_rows, c), lambda i: (i, 0))],
        out_specs=[
            pl.BlockSpec((block_rows, c), lambda i: (i, 0)),
            pl.BlockSpec((ld_block,), lambda i: (i,)),
        ],
        out_shape=[
            jax.ShapeDtypeStruct((n, c), inputs.dtype),
            jax.ShapeDtypeStruct((n,), jnp.float32),
        ],
        compiler_params=pltpu.CompilerParams(
            dimension_semantics=("parallel",),
            vmem_limit_bytes=128 * 1024 * 1024,
        ),
    )(inputs)
    return (outputs, ld)
